# Initial kernel scaffold; baseline (speedup 1.0000x reference)
import jax
import jax.numpy as jnp
from jax.experimental import pallas as pl


def kernel(x, edge_index, W1, b1, W2, b2, W_lin, b_lin):
    raise NotImplementedError("write your pallas kernel here")


# speedup vs baseline: 9.3350x; 9.3350x over previous
import functools

import jax
import jax.numpy as jnp
from jax import lax
from jax.experimental import pallas as pl
from jax.experimental.pallas import tpu as pltpu
from jax.experimental.pallas import tpu_sc as plsc

N = 10000
E = 320000
D = 128
NCLS = 40
NC, NS = 2, 16
R = 10240
EPT = E // (NC * NS)
K = 128
CH = 80
STRIPE = R // NS
BM = 1000

_mesh = plsc.VectorSubcoreMesh(
    core_axis_name="c", subcore_axis_name="s", num_cores=NC, num_subcores=NS)


@functools.partial(
    pl.kernel,
    out_type=jax.ShapeDtypeStruct((NC, R, D), jnp.float32),
    mesh=_mesh,
    scratch_types=[
        pltpu.VMEM_SHARED((R, D), jnp.float32),
        pltpu.VMEM((CH, K), jnp.int32),
        pltpu.VMEM((K, D), jnp.float32),
        pltpu.VMEM((K, D), jnp.float32),
    ],
)
def _deg_kernel(dstp_hbm, ones_hbm, zeros_hbm, degp_hbm, acc, dst_idx, ones_v, zeros_v):
    cid = lax.axis_index("c")
    sid = lax.axis_index("s")
    base = sid * STRIPE
    pltpu.sync_copy(zeros_hbm, zeros_v)
    for kk in range(STRIPE // K):
        pltpu.sync_copy(zeros_v, acc.at[pl.ds(base + kk * K, K)])
    plsc.subcore_barrier()
    pltpu.sync_copy(ones_hbm, ones_v)
    pltpu.sync_copy(dstp_hbm.at[cid, sid], dst_idx)

    def body(j, carry):
        pltpu.sync_copy(ones_v, acc.at[dst_idx.at[j]], add=True)
        return carry

    lax.fori_loop(0, CH, body, 0)
    plsc.subcore_barrier()
    for kk in range(STRIPE // K):
        pltpu.sync_copy(acc.at[pl.ds(base + kk * K, K)],
                        degp_hbm.at[cid, pl.ds(base + kk * K, K)])


@functools.partial(
    pl.kernel,
    out_type=jax.ShapeDtypeStruct((NC, R, D), jnp.float32),
    mesh=_mesh,
    scratch_types=[
        pltpu.VMEM_SHARED((R, D), jnp.float32),
        pltpu.VMEM((CH, K), jnp.int32),
        pltpu.VMEM((CH, K), jnp.int32),
        pltpu.VMEM((K, D), jnp.float32),
        pltpu.SemaphoreType.DMA,
    ],
)
def _agg_kernel(ht_hbm, srcp_hbm, dstp_hbm, zrows_hbm, aggp_hbm,
                acc, src_idx, dst_idx, rows, sem):
    cid = lax.axis_index("c")
    sid = lax.axis_index("s")
    base = sid * STRIPE
    pltpu.sync_copy(zrows_hbm, rows)
    for kk in range(STRIPE // K):
        pltpu.sync_copy(rows, acc.at[pl.ds(base + kk * K, K)])
    plsc.subcore_barrier()
    pltpu.sync_copy(srcp_hbm.at[cid, sid], src_idx)
    pltpu.sync_copy(dstp_hbm.at[cid, sid], dst_idx)

    def body(j, carry):
        pltpu.async_copy(ht_hbm.at[src_idx.at[j]], rows, sem).wait()
        pltpu.sync_copy(rows, acc.at[dst_idx.at[j]], add=True)
        return carry

    lax.fori_loop(0, CH, body, 0)
    plsc.subcore_barrier()
    for kk in range(STRIPE // K):
        pltpu.sync_copy(acc.at[pl.ds(base + kk * K, K)],
                        aggp_hbm.at[cid, pl.ds(base + kk * K, K)])


def _dinv_of(degp_blk):
    d = degp_blk[0, :, 0:1] + degp_blk[1, :, 0:1] + 1.0
    return lax.rsqrt(d)


def _tc1_body(degp, x, W1, ht_out):
    dinv = _dinv_of(degp[...])
    ht_out[...] = jnp.dot(x[...], W1[...],
                          preferred_element_type=jnp.float32) * dinv


def _tc2_body(degp, aggp, ht1, W2, b1, ht2_out):
    dinv = _dinv_of(degp[...])
    a = aggp[...]
    comb = dinv * (a[0] + a[1] + ht1[...]) + b1[...]
    h = jnp.maximum(comb, 0.0)
    ht2_out[...] = jnp.dot(h, W2[...],
                           preferred_element_type=jnp.float32) * dinv


def _tc3_body(degp, aggp, ht2, W3, b2, b3, out):
    dinv = _dinv_of(degp[...])
    a = aggp[...]
    h = jnp.maximum(dinv * (a[0] + a[1] + ht2[...]) + b2[...], 0.0)
    logits = jnp.dot(h, W3[...], preferred_element_type=jnp.float32) + b3[...]
    m = jnp.max(logits, axis=-1, keepdims=True)
    lse = m + jnp.log(jnp.sum(jnp.exp(logits - m), axis=-1, keepdims=True))
    out[...] = logits - lse


_deg_spec = pl.BlockSpec((NC, BM, D), lambda i: (0, i, 0))
_agg_spec = pl.BlockSpec((NC, BM, D), lambda i: (0, i, 0))
_row_spec = pl.BlockSpec((BM, D), lambda i: (i, 0))
_mat_spec = pl.BlockSpec((D, D), lambda i: (0, 0))
_vec_spec = pl.BlockSpec((1, D), lambda i: (0, 0))

_tc1 = pl.pallas_call(
    _tc1_body,
    grid=(N // BM,),
    in_specs=[_deg_spec, _row_spec, _mat_spec],
    out_specs=_row_spec,
    out_shape=jax.ShapeDtypeStruct((N, D), jnp.float32),
)

_tc2 = pl.pallas_call(
    _tc2_body,
    grid=(N // BM,),
    in_specs=[_deg_spec, _agg_spec, _row_spec, _mat_spec, _vec_spec],
    out_specs=_row_spec,
    out_shape=jax.ShapeDtypeStruct((N, D), jnp.float32),
)

_tc3 = pl.pallas_call(
    _tc3_body,
    grid=(N // BM,),
    in_specs=[_deg_spec, _agg_spec, _row_spec, _mat_spec, _vec_spec, _vec_spec],
    out_specs=_row_spec,
    out_shape=jax.ShapeDtypeStruct((N, D), jnp.float32),
)


def kernel(x, edge_index, W1, b1, W2, b2, W_lin, b_lin):
    src = edge_index[0]
    dst = edge_index[1]
    pad = CH * K - EPT
    srcp = jnp.pad(src.reshape(NC, NS, EPT), ((0, 0), (0, 0), (0, pad)),
                   constant_values=0).reshape(NC, NS, CH, K)
    dstp = jnp.pad(dst.reshape(NC, NS, EPT), ((0, 0), (0, 0), (0, pad)),
                   constant_values=N).reshape(NC, NS, CH, K)
    onesKD = jnp.ones((K, D), jnp.float32)
    zrows = jnp.zeros((K, D), jnp.float32)
    W3 = jnp.pad(W_lin, ((0, 0), (0, D - NCLS)))
    b3 = jnp.pad(b_lin, (0, D - NCLS), constant_values=-1e30).reshape(1, D)
    b1r = b1.reshape(1, D)
    b2r = b2.reshape(1, D)

    degp = _deg_kernel(dstp, onesKD, zrows)
    ht1 = _tc1(degp, x, W1)
    a1 = _agg_kernel(ht1, srcp, dstp, zrows)
    ht2 = _tc2(degp, a1, ht1, W2, b1r)
    a2 = _agg_kernel(ht2, srcp, dstp, zrows)
    logp = _tc3(degp, a2, ht2, W3, b2r, b3)
    return logp[:, :NCLS]


# speedup vs baseline: 10.5527x; 1.1305x over previous
import functools

import jax
import jax.numpy as jnp
from jax import lax
from jax.experimental import pallas as pl
from jax.experimental.pallas import tpu as pltpu
from jax.experimental.pallas import tpu_sc as plsc

N = 10000
E = 320000
D = 128
NCLS = 40
NC, NS = 2, 16
R = 10240
EPT = E // (NC * NS)
K = 128
CH = 80
STRIPE = R // NS
BM = 1000

_mesh = plsc.VectorSubcoreMesh(
    core_axis_name="c", subcore_axis_name="s", num_cores=NC, num_subcores=NS)


@functools.partial(
    pl.kernel,
    out_type=jax.ShapeDtypeStruct((NC, R, D), jnp.float32),
    mesh=_mesh,
    scratch_types=[
        pltpu.VMEM_SHARED((R, D), jnp.float32),
        pltpu.VMEM((CH, K), jnp.int32),
        pltpu.VMEM((K, D), jnp.float32),
        pltpu.VMEM((K, D), jnp.float32),
    ],
)
def _deg_kernel(dstp_hbm, ones_hbm, zeros_hbm, degp_hbm, acc, dst_idx, ones_v, zeros_v):
    cid = lax.axis_index("c")
    sid = lax.axis_index("s")
    base = sid * STRIPE
    pltpu.sync_copy(zeros_hbm, zeros_v)
    for kk in range(STRIPE // K):
        pltpu.sync_copy(zeros_v, acc.at[pl.ds(base + kk * K, K)])
    plsc.subcore_barrier()
    pltpu.sync_copy(ones_hbm, ones_v)
    pltpu.sync_copy(dstp_hbm.at[cid, sid], dst_idx)

    def body(j, carry):
        pltpu.sync_copy(ones_v, acc.at[dst_idx.at[j]], add=True)
        return carry

    lax.fori_loop(0, CH, body, 0)
    plsc.subcore_barrier()
    for kk in range(STRIPE // K):
        pltpu.sync_copy(acc.at[pl.ds(base + kk * K, K)],
                        degp_hbm.at[cid, pl.ds(base + kk * K, K)])


@functools.partial(
    pl.kernel,
    out_type=jax.ShapeDtypeStruct((NC, R, D), jnp.float32),
    mesh=_mesh,
    scratch_types=[
        pltpu.VMEM_SHARED((R, D), jnp.float32),
        pltpu.VMEM((CH // 2, K), jnp.int32),
        pltpu.VMEM((CH // 2, K), jnp.int32),
        pltpu.VMEM((K, D), jnp.float32),
        pltpu.VMEM((K, D), jnp.float32),
        pltpu.SemaphoreType.DMA,
        pltpu.SemaphoreType.DMA,
    ],
)
def _agg_kernel(ht_hbm, srcp_hbm, dstp_hbm, zrows_hbm, aggp_hbm,
                acc, src_idx, dst_idx, rows0, rows1, sem0, sem1):
    cid = lax.axis_index("c")
    sid = lax.axis_index("s")
    base = sid * STRIPE
    HCH = CH // 2
    pltpu.sync_copy(zrows_hbm, rows0)
    for kk in range(STRIPE // K):
        pltpu.sync_copy(rows0, acc.at[pl.ds(base + kk * K, K)])
    plsc.subcore_barrier()

    for p in range(2):
        pltpu.sync_copy(srcp_hbm.at[cid, sid, pl.ds(p * HCH, HCH)], src_idx)
        pltpu.sync_copy(dstp_hbm.at[cid, sid, pl.ds(p * HCH, HCH)], dst_idx)
        pltpu.async_copy(ht_hbm.at[src_idx.at[0]], rows0, sem0)

        def body(j2, carry):
            j = 2 * j2
            pltpu.async_copy(ht_hbm.at[src_idx.at[j + 1]], rows1, sem1)
            pltpu.make_async_copy(ht_hbm.at[src_idx.at[j]], rows0, sem0).wait()
            pltpu.sync_copy(rows0, acc.at[dst_idx.at[j]], add=True)

            @pl.when(j2 < HCH // 2 - 1)
            def _():
                pltpu.async_copy(ht_hbm.at[src_idx.at[j + 2]], rows0, sem0)

            pltpu.make_async_copy(ht_hbm.at[src_idx.at[j + 1]], rows1, sem1).wait()
            pltpu.sync_copy(rows1, acc.at[dst_idx.at[j + 1]], add=True)
            return carry

        lax.fori_loop(0, HCH // 2, body, 0)
    plsc.subcore_barrier()
    for kk in range(STRIPE // K):
        pltpu.sync_copy(acc.at[pl.ds(base + kk * K, K)],
                        aggp_hbm.at[cid, pl.ds(base + kk * K, K)])


def _dinv_of(degp_blk):
    d = degp_blk[0, :, 0:1] + degp_blk[1, :, 0:1] + 1.0
    return lax.rsqrt(d)


def _tc1_body(degp, x, W1, ht_out):
    dinv = _dinv_of(degp[...])
    ht_out[...] = jnp.dot(x[...], W1[...],
                          preferred_element_type=jnp.float32) * dinv


def _tc2_body(degp, aggp, ht1, W2, b1, ht2_out):
    dinv = _dinv_of(degp[...])
    a = aggp[...]
    comb = dinv * (a[0] + a[1] + ht1[...]) + b1[...]
    h = jnp.maximum(comb, 0.0)
    ht2_out[...] = jnp.dot(h, W2[...],
                           preferred_element_type=jnp.float32) * dinv


def _tc3_body(degp, aggp, ht2, W3, b2, b3, out):
    dinv = _dinv_of(degp[...])
    a = aggp[...]
    h = jnp.maximum(dinv * (a[0] + a[1] + ht2[...]) + b2[...], 0.0)
    logits = jnp.dot(h, W3[...], preferred_element_type=jnp.float32) + b3[...]
    m = jnp.max(logits, axis=-1, keepdims=True)
    lse = m + jnp.log(jnp.sum(jnp.exp(logits - m), axis=-1, keepdims=True))
    out[...] = logits - lse


_deg_spec = pl.BlockSpec((NC, BM, D), lambda i: (0, i, 0))
_agg_spec = pl.BlockSpec((NC, BM, D), lambda i: (0, i, 0))
_row_spec = pl.BlockSpec((BM, D), lambda i: (i, 0))
_mat_spec = pl.BlockSpec((D, D), lambda i: (0, 0))
_vec_spec = pl.BlockSpec((1, D), lambda i: (0, 0))

_tc1 = pl.pallas_call(
    _tc1_body,
    grid=(N // BM,),
    in_specs=[_deg_spec, _row_spec, _mat_spec],
    out_specs=_row_spec,
    out_shape=jax.ShapeDtypeStruct((N, D), jnp.float32),
)

_tc2 = pl.pallas_call(
    _tc2_body,
    grid=(N // BM,),
    in_specs=[_deg_spec, _agg_spec, _row_spec, _mat_spec, _vec_spec],
    out_specs=_row_spec,
    out_shape=jax.ShapeDtypeStruct((N, D), jnp.float32),
)

_tc3 = pl.pallas_call(
    _tc3_body,
    grid=(N // BM,),
    in_specs=[_deg_spec, _agg_spec, _row_spec, _mat_spec, _vec_spec, _vec_spec],
    out_specs=_row_spec,
    out_shape=jax.ShapeDtypeStruct((N, D), jnp.float32),
)


def kernel(x, edge_index, W1, b1, W2, b2, W_lin, b_lin):
    src = edge_index[0]
    dst = edge_index[1]
    pad = CH * K - EPT
    srcp = jnp.pad(src.reshape(NC, NS, EPT), ((0, 0), (0, 0), (0, pad)),
                   constant_values=0).reshape(NC, NS, CH, K)
    dstp = jnp.pad(dst.reshape(NC, NS, EPT), ((0, 0), (0, 0), (0, pad)),
                   constant_values=N).reshape(NC, NS, CH, K)
    onesKD = jnp.ones((K, D), jnp.float32)
    zrows = jnp.zeros((K, D), jnp.float32)
    W3 = jnp.pad(W_lin, ((0, 0), (0, D - NCLS)))
    b3 = jnp.pad(b_lin, (0, D - NCLS), constant_values=-1e30).reshape(1, D)
    b1r = b1.reshape(1, D)
    b2r = b2.reshape(1, D)

    degp = _deg_kernel(dstp, onesKD, zrows)
    ht1 = _tc1(degp, x, W1)
    a1 = _agg_kernel(ht1, srcp, dstp, zrows)
    ht2 = _tc2(degp, a1, ht1, W2, b1r)
    a2 = _agg_kernel(ht2, srcp, dstp, zrows)
    logp = _tc3(degp, a2, ht2, W3, b2r, b3)
    return logp[:, :NCLS]
